# baseline (speedup 1.0000x reference)
import functools

import jax
import jax.numpy as jnp
from jax import lax
from jax.experimental import pallas as pl
from jax.experimental.pallas import tpu as pltpu
from jax.experimental.pallas import tpu_sc as plsc

BS, SEQ = 8, 50
B = BS * SEQ
N = 200
E = 3200
D = 64
C1 = 111
HID = 128
FEAT = 177
LAYERS = 4

NC, NS, L = 2, 16, 16
NW = NC * NS
ROWS = B * N
RPW = 2560
ROWS_PAD = RPW * NW
GCHUNK = 128
NGC = RPW // GCHUNK
GPW = (B + NW - 1) // NW

GB = 16
GRID1 = B // GB
NP = 208


def _sig(x):
    return 0.5 * jnp.tanh(0.5 * x) + 0.5


def _sc_body(nid, table, src, dst, et, eew,
             x0, a_out,
             idx_v0, idx_v1, rows_v0, rows_v1, eew_v, ew8_v,
             sv0, sv1, dv0, dv1, tv0, tv1, av0, av1,
             isem0, isem1, gsem0, gsem1, osem0, osem1,
             esem0, esem1, asem0, asem1):
    cid = lax.axis_index("c")
    sid = lax.axis_index("s")
    wid = sid * NC + cid
    lanes = lax.broadcasted_iota(jnp.int32, (L,), 0)

    iv = [idx_v0, idx_v1]
    rv = [rows_v0, rows_v1]
    sv = [sv0, sv1]
    dv = [dv0, dv1]
    tv = [tv0, tv1]
    av = [av0, av1]
    isem = [isem0, isem1]
    gsem = [gsem0, gsem1]
    osem = [osem0, osem1]
    esem = [esem0, esem1]
    asem = [asem0, asem1]

    def idxd(c):
        base = wid * RPW + c * GCHUNK
        return pltpu.make_async_copy(nid.at[pl.ds(base, GCHUNK)],
                                     iv[c % 2], isem[c % 2])

    def gatd(c):
        return pltpu.make_async_copy(table.at[iv[c % 2]], rv[c % 2],
                                     gsem[c % 2])

    def outd(c):
        base = wid * RPW + c * GCHUNK
        return pltpu.make_async_copy(rv[c % 2],
                                     x0.at[pl.ds(base, GCHUNK)],
                                     osem[c % 2])

    def edged(k, field):
        b = k % 2
        g = wid + k * NW
        srcs = {"s": (src, sv), "d": (dst, dv), "t": (et, tv)}
        hbm, bufs = srcs[field]
        return pltpu.make_async_copy(hbm.at[g], bufs[b], esem[b])

    def start_edges(k):
        g = wid + k * NW

        @pl.when(g < B)
        def _():
            edged(k, "s").start()
            edged(k, "d").start()
            edged(k, "t").start()

    def ad(k):
        b = k % 2
        g = wid + k * NW
        return pltpu.make_async_copy(av[b], a_out.at[pl.ds(g * N, N)],
                                     asem[b])

    start_edges(0)
    pltpu.sync_copy(eew, eew_v)

    idxd(0).start()
    idxd(1).start()
    idxd(0).wait()
    gatd(0).start()
    for c in range(NGC):
        if c + 1 < NGC:
            idxd(c + 1).wait()
            if c >= 1:
                outd(c - 1).wait()
            gatd(c + 1).start()
        gatd(c).wait()
        outd(c).start()
        if c + 2 < NGC:
            idxd(c + 2).start()
    outd(NGC - 2).wait()
    outd(NGC - 1).wait()

    base8 = jnp.minimum(lanes, 7) * D
    acc = jnp.zeros((L,), jnp.float32)
    for k in range(D):
        acc = acc + plsc.load_gather(eew_v, [base8 + k])
    ew8_v[...] = acc * (1.0 / D)

    masks = [lanes == l for l in range(L)]

    for k in range(GPW):
        b = k % 2
        g = wid + k * NW
        if k + 1 < GPW:
            start_edges(k + 1)

        @pl.when(g < B)
        def _(k=k, b=b, g=g):
            if k >= 2:
                ad(k - 2).wait()

            def zero_loop(j, c2):
                for u in range(NP // L):
                    av[b][j, pl.ds(u * L, L)] = jnp.zeros((L,), jnp.float32)
                return c2

            lax.fori_loop(0, N, zero_loop, 0)
            edged(k, "s").wait()
            edged(k, "d").wait()
            edged(k, "t").wait()

            def edge_loop(c, c2):
                for u in range(2):
                    o = (c * 2 + u) * L
                    s = sv[b][pl.ds(o, L)]
                    d = dv[b][pl.ds(o, L)]
                    t = tv[b][pl.ds(o, L)]
                    w = plsc.load_gather(ew8_v, [t])
                    for m in masks:
                        plsc.addupdate_scatter(av[b], [d, s], w, mask=m)
                return c2

            lax.fori_loop(0, E // (2 * L), edge_loop, 0)
            ad(k).start()

    for k in (GPW - 2, GPW - 1):
        g = wid + k * NW

        @pl.when(g < B)
        def _(k=k):
            ad(k).wait()


@jax.jit
def _sc_build(nid, table, src, dst, et, eew):
    mesh = plsc.VectorSubcoreMesh(core_axis_name="c", subcore_axis_name="s",
                                  num_cores=NC, num_subcores=NS)
    fn = pl.kernel(
        _sc_body,
        out_type=[jax.ShapeDtypeStruct((ROWS_PAD, D), jnp.float32),
                  jax.ShapeDtypeStruct((B * N, NP), jnp.float32)],
        mesh=mesh,
        scratch_types=(
            [pltpu.VMEM((GCHUNK,), jnp.int32)] * 2
            + [pltpu.VMEM((GCHUNK, D), jnp.float32)] * 2
            + [pltpu.VMEM((8 * D,), jnp.float32),
               pltpu.VMEM((L,), jnp.float32)]
            + [pltpu.VMEM((E,), jnp.int32)] * 6
            + [pltpu.VMEM((N, NP), jnp.float32)] * 2
            + [pltpu.SemaphoreType.DMA] * 10
        ),
        compiler_params=pltpu.CompilerParams(needs_layout_passes=False,
                                             use_tc_tiling_on_sc=False),
    )
    return fn(nid, table, src, dst, et, eew)


def _ggnn_body(x0_ref, a_ref, ce_ref, cid_ref, cemb_ref, gw_ref,
               wcat_ref, wnh_ref, bsum_ref, bhn_ref, out_ref):
    cemb = cemb_ref[...]
    wcat = wcat_ref[...]
    wnh = wnh_ref[...]
    bsum = bsum_ref[0]
    bhn = bhn_ref[0]
    f32 = jnp.float32
    bf = jnp.bfloat16

    a_bf = [a_ref[pl.ds(g * N, N), 0:N].astype(bf) for g in range(GB)]
    xf = x0_ref[...]
    for i in range(LAYERS):
        w = gw_ref[i]
        xb = xf.astype(bf)
        mf = jnp.dot(xb, w, preferred_element_type=f32)
        m3 = mf.astype(bf).reshape(GB, N, D)
        aggs = [jnp.dot(a_bf[g], m3[g], preferred_element_type=f32)
                for g in range(GB)]
        aggf = jnp.concatenate(aggs, axis=0)
        cat = jnp.concatenate([aggf, xf], axis=1).astype(bf)
        p = jnp.dot(cat, wcat, preferred_element_type=f32)
        hn = jnp.dot(xb, wnh, preferred_element_type=f32) + bhn
        r = _sig(p[:, 0:D] + bsum[0:D])
        z = _sig(p[:, 128:128 + D] + bsum[D:2 * D])
        n = jnp.tanh(p[:, 256:256 + D] + bsum[2 * D:] - (1.0 - r) * hn)
        xf = (1.0 - z) * n + z * xf

    x3 = xf.astype(bf).reshape(GB, N, D)
    for g in range(GB):
        ce = ce_ref[g]
        xg = x3[g]
        q = (ce[:, None] * cemb).astype(bf)
        sc = lax.dot_general(q, xg, (((1,), (1,)), ((), ())),
                             preferred_element_type=f32)
        ex = jnp.exp(sc)
        aw = (ex / jnp.sum(ex, axis=1, keepdims=True)).astype(bf)
        ao = jnp.dot(aw, xg, preferred_element_type=f32) * ce[:, None]
        nsum = jnp.sum(cid_ref[g])
        nsum = jnp.where(nsum == 0.0, 1.0, nsum)
        out_ref[g] = jnp.sum(ao, axis=0) / nsum


@jax.jit
def _tc_ggnn(x0, a3, ce, cid, cemb, gw, wcat, wnh, bsum, bhn):
    return pl.pallas_call(
        _ggnn_body,
        grid=(GRID1,),
        in_specs=[
            pl.BlockSpec((GB * N, D), lambda i: (i, 0)),
            pl.BlockSpec((GB * N, NP), lambda i: (i, 0)),
            pl.BlockSpec((GB, C1), lambda i: (i, 0)),
            pl.BlockSpec((GB, C1), lambda i: (i, 0)),
            pl.BlockSpec((C1, D), lambda i: (0, 0)),
            pl.BlockSpec((LAYERS, D, D), lambda i: (0, 0, 0)),
            pl.BlockSpec((2 * D, 384), lambda i: (0, 0)),
            pl.BlockSpec((D, D), lambda i: (0, 0)),
            pl.BlockSpec((1, 3 * D), lambda i: (0, 0)),
            pl.BlockSpec((1, D), lambda i: (0, 0)),
        ],
        out_specs=pl.BlockSpec((GB, D), lambda i: (i, 0)),
        out_shape=jax.ShapeDtypeStruct((B, D), jnp.float32),
        compiler_params=pltpu.CompilerParams(
            dimension_semantics=("arbitrary",)),
    )(x0, a3, ce, cid, cemb, gw, wcat, wnh, bsum, bhn)


def _lstm_body(ce_ref, at_ref, cr_ref, tc_ref, res_ref,
               wih_ref, whh_ref, bih_ref, bhh_ref, pw_ref, pb_ref,
               loss_ref, sig_ref, ft_ref, ihs_ref, hs_ref):
    f32 = jnp.float32
    ce3 = jnp.transpose(ce_ref[...], (1, 0, 2))
    at3 = jnp.transpose(at_ref[...].reshape(BS, SEQ, D), (1, 0, 2))
    cr3 = jnp.transpose(cr_ref[...], (1, 0, 2))
    xcat = jnp.concatenate([ce3, at3, cr3], axis=2)
    xflat = xcat.reshape(SEQ * BS, FEAT)
    ih = lax.dot_general(xflat, wih_ref[...], (((1,), (1,)), ((), ())),
                         preferred_element_type=f32) + bih_ref[0]
    ihs_ref[...] = ih.reshape(SEQ, BS, 4 * HID)
    whh = whh_ref[...]
    bhh = bhh_ref[0]

    def step(t, hc):
        h, c = hc
        g = ihs_ref[t] + lax.dot_general(
            h, whh, (((1,), (1,)), ((), ())), preferred_element_type=f32) + bhh
        i_g = g[:, 0:HID]
        f_g = g[:, HID:2 * HID]
        g_g = g[:, 2 * HID:3 * HID]
        o_g = g[:, 3 * HID:]
        c = _sig(f_g) * c + _sig(i_g) * jnp.tanh(g_g)
        h = _sig(o_g) * jnp.tanh(c)
        hs_ref[pl.ds(t, 1)] = h.reshape(1, BS, HID)
        return (h, c)

    lax.fori_loop(0, SEQ, step,
                  (jnp.zeros((BS, HID), f32), jnp.zeros((BS, HID), f32)))

    lo = hs_ref[...].reshape(SEQ * BS, HID)
    pred = lax.dot_general(lo, pw_ref[...], (((1,), (1,)), ((), ())),
                           preferred_element_type=f32) + pb_ref[0]
    tc = jnp.transpose(tc_ref[...], (1, 0, 2))
    p1 = jnp.sum(pred.reshape(SEQ, BS, C1) * tc, axis=2)
    numc = jnp.sum(tc, axis=2)
    mask = numc > 0.0
    safe = jnp.where(mask, numc, 1.0)
    fp = p1 / safe
    ft = res_ref[...]
    losses = (jnp.maximum(fp, 0.0) - fp * ft +
              jnp.log(1.0 + jnp.exp(-jnp.abs(fp))))
    cnt = jnp.sum(mask.astype(f32))
    loss = jnp.sum(jnp.where(mask, losses, 0.0)) / jnp.maximum(cnt, 1.0)
    loss_ref[...] = jnp.broadcast_to(loss, (1, 1))
    sig_ref[...] = _sig(jnp.where(mask, fp, 0.0))
    ft_ref[...] = jnp.where(mask, ft, 0.0)


@jax.jit
def _tc_lstm(ce_t, at_t, cr_t, tc_t, res_t, wih, whh, bih, bhh, pw, pb):
    return pl.pallas_call(
        _lstm_body,
        out_shape=[jax.ShapeDtypeStruct((1, 1), jnp.float32),
                   jax.ShapeDtypeStruct((SEQ, BS), jnp.float32),
                   jax.ShapeDtypeStruct((SEQ, BS), jnp.float32)],
        scratch_shapes=[pltpu.VMEM((SEQ, BS, 4 * HID), jnp.float32),
                        pltpu.VMEM((SEQ, BS, HID), jnp.float32)],
    )(ce_t, at_t, cr_t, tc_t, res_t, wih, whh, bih, bhh, pw, pb)


def kernel(p_id, c_id, node_id, edge, edge_type, target_c, result, c_embed,
           cur_result, node_embed_w, edge_embed_w, ggnn_w, gru_w_ih, gru_w_hh,
           gru_b_ih, gru_b_hh, lstm_w_ih, lstm_w_hh, lstm_b_ih, lstm_b_hh,
           pred_w, pred_b, concept_embedding):
    f32 = jnp.float32
    nid = node_id.reshape(ROWS).astype(jnp.int32)
    nid = jnp.concatenate(
        [nid, jnp.zeros((ROWS_PAD - ROWS,), jnp.int32)])
    src = edge[:, :, 0, :].reshape(B, E).astype(jnp.int32)
    dst = edge[:, :, 1, :].reshape(B, E).astype(jnp.int32)
    et = edge_type.reshape(B, E).astype(jnp.int32)

    x0_rows, a_mat = _sc_build(nid, node_embed_w.astype(f32), src, dst, et,
                               edge_embed_w.reshape(8 * D).astype(f32))

    bf = jnp.bfloat16
    wt = jnp.concatenate([gru_w_ih.T, gru_w_hh.T], axis=0)
    zpad = jnp.zeros((2 * D, D), f32)
    wcat = jnp.concatenate([wt[:, 0:D], zpad, wt[:, D:2 * D], zpad,
                            wt[:, 2 * D:], zpad], axis=1)
    wnh = gru_w_hh[2 * D:].T
    bsum = (gru_b_ih + gru_b_hh).reshape(1, 3 * D)
    bhn = gru_b_hh[2 * D:].reshape(1, D)
    attn = _tc_ggnn(x0_rows, a_mat, c_embed.reshape(B, C1).astype(f32),
                    c_id.reshape(B, C1).astype(f32),
                    concept_embedding.astype(bf), ggnn_w.astype(bf),
                    wcat.astype(bf), wnh.astype(bf),
                    bsum.astype(f32), bhn.astype(f32))

    res_t = result.reshape(BS, SEQ).T.astype(f32)

    loss2, sig2, ft2 = _tc_lstm(
        c_embed.astype(f32), attn, cur_result.astype(f32),
        target_c.astype(f32), res_t, lstm_w_ih.astype(f32),
        lstm_w_hh.astype(f32), lstm_b_ih.reshape(1, 4 * HID).astype(f32),
        lstm_b_hh.reshape(1, 4 * HID).astype(f32), pred_w.astype(f32),
        pred_b.reshape(1, C1).astype(f32))
    return (loss2.reshape(()), sig2.T.reshape(B), ft2.T.reshape(B))


# speedup vs baseline: 1.0111x; 1.0111x over previous
import functools

import jax
import jax.numpy as jnp
from jax import lax
from jax.experimental import pallas as pl
from jax.experimental.pallas import tpu as pltpu
from jax.experimental.pallas import tpu_sc as plsc

BS, SEQ = 8, 50
B = BS * SEQ
N = 200
E = 3200
D = 64
C1 = 111
HID = 128
FEAT = 177
LAYERS = 4

NC, NS, L = 2, 16, 16
NW = NC * NS
ROWS = B * N
RPW = 2560
ROWS_PAD = RPW * NW
GCHUNK = 128
NGC = RPW // GCHUNK
GPW = (B + NW - 1) // NW

GB = 8
GRID1 = B // GB
NP = 208


def _sig(x):
    return 0.5 * jnp.tanh(0.5 * x) + 0.5


def _sc_body(nid, table, src, dst, et, eew,
             x0, a_out,
             idx_v0, idx_v1, rows_v0, rows_v1, eew_v, ew8_v,
             sv0, sv1, dv0, dv1, tv0, tv1, av0, av1,
             isem0, isem1, gsem0, gsem1, osem0, osem1,
             esem0, esem1, asem0, asem1):
    cid = lax.axis_index("c")
    sid = lax.axis_index("s")
    wid = sid * NC + cid
    lanes = lax.broadcasted_iota(jnp.int32, (L,), 0)

    iv = [idx_v0, idx_v1]
    rv = [rows_v0, rows_v1]
    sv = [sv0, sv1]
    dv = [dv0, dv1]
    tv = [tv0, tv1]
    av = [av0, av1]
    isem = [isem0, isem1]
    gsem = [gsem0, gsem1]
    osem = [osem0, osem1]
    esem = [esem0, esem1]
    asem = [asem0, asem1]

    def idxd(c):
        base = wid * RPW + c * GCHUNK
        return pltpu.make_async_copy(nid.at[pl.ds(base, GCHUNK)],
                                     iv[c % 2], isem[c % 2])

    def gatd(c):
        return pltpu.make_async_copy(table.at[iv[c % 2]], rv[c % 2],
                                     gsem[c % 2])

    def outd(c):
        base = wid * RPW + c * GCHUNK
        return pltpu.make_async_copy(rv[c % 2],
                                     x0.at[pl.ds(base, GCHUNK)],
                                     osem[c % 2])

    def edged(k, field):
        b = k % 2
        g = wid + k * NW
        srcs = {"s": (src, sv), "d": (dst, dv), "t": (et, tv)}
        hbm, bufs = srcs[field]
        return pltpu.make_async_copy(hbm.at[g], bufs[b], esem[b])

    def start_edges(k):
        g = wid + k * NW

        @pl.when(g < B)
        def _():
            edged(k, "s").start()
            edged(k, "d").start()
            edged(k, "t").start()

    def ad(k):
        b = k % 2
        g = wid + k * NW
        return pltpu.make_async_copy(av[b], a_out.at[pl.ds(g * N, N)],
                                     asem[b])

    start_edges(0)
    pltpu.sync_copy(eew, eew_v)

    idxd(0).start()
    idxd(1).start()
    idxd(0).wait()
    gatd(0).start()
    for c in range(NGC):
        if c + 1 < NGC:
            idxd(c + 1).wait()
            if c >= 1:
                outd(c - 1).wait()
            gatd(c + 1).start()
        gatd(c).wait()
        outd(c).start()
        if c + 2 < NGC:
            idxd(c + 2).start()
    outd(NGC - 2).wait()
    outd(NGC - 1).wait()

    base8 = jnp.minimum(lanes, 7) * D
    acc = jnp.zeros((L,), jnp.float32)
    for k in range(D):
        acc = acc + plsc.load_gather(eew_v, [base8 + k])
    ew8_v[...] = acc * (1.0 / D)

    masks = [lanes == l for l in range(L)]

    for k in range(GPW):
        b = k % 2
        g = wid + k * NW
        if k + 1 < GPW:
            start_edges(k + 1)

        @pl.when(g < B)
        def _(k=k, b=b, g=g):
            if k >= 2:
                ad(k - 2).wait()

            def zero_loop(j, c2):
                for u in range(NP // L):
                    av[b][j, pl.ds(u * L, L)] = jnp.zeros((L,), jnp.float32)
                return c2

            lax.fori_loop(0, N, zero_loop, 0)
            edged(k, "s").wait()
            edged(k, "d").wait()
            edged(k, "t").wait()

            def edge_loop(c, c2):
                for u in range(2):
                    o = (c * 2 + u) * L
                    s = sv[b][pl.ds(o, L)]
                    d = dv[b][pl.ds(o, L)]
                    t = tv[b][pl.ds(o, L)]
                    w = plsc.load_gather(ew8_v, [t])
                    for m in masks:
                        plsc.addupdate_scatter(av[b], [d, s], w, mask=m)
                return c2

            lax.fori_loop(0, E // (2 * L), edge_loop, 0)
            ad(k).start()

    for k in (GPW - 2, GPW - 1):
        g = wid + k * NW

        @pl.when(g < B)
        def _(k=k):
            ad(k).wait()


@jax.jit
def _sc_build(nid, table, src, dst, et, eew):
    mesh = plsc.VectorSubcoreMesh(core_axis_name="c", subcore_axis_name="s",
                                  num_cores=NC, num_subcores=NS)
    fn = pl.kernel(
        _sc_body,
        out_type=[jax.ShapeDtypeStruct((ROWS_PAD, D), jnp.float32),
                  jax.ShapeDtypeStruct((B * N, NP), jnp.float32)],
        mesh=mesh,
        scratch_types=(
            [pltpu.VMEM((GCHUNK,), jnp.int32)] * 2
            + [pltpu.VMEM((GCHUNK, D), jnp.float32)] * 2
            + [pltpu.VMEM((8 * D,), jnp.float32),
               pltpu.VMEM((L,), jnp.float32)]
            + [pltpu.VMEM((E,), jnp.int32)] * 6
            + [pltpu.VMEM((N, NP), jnp.float32)] * 2
            + [pltpu.SemaphoreType.DMA] * 10
        ),
        compiler_params=pltpu.CompilerParams(needs_layout_passes=False,
                                             use_tc_tiling_on_sc=False),
    )
    return fn(nid, table, src, dst, et, eew)


def _ggnn_body(x0_ref, a_ref, ce_ref, cid_ref, cemb_ref, gw_ref,
               wcat_ref, wnh_ref, bsum_ref, bhn_ref, out_ref):
    cemb = cemb_ref[...]
    wcat = wcat_ref[...]
    wnh = wnh_ref[...]
    bsum = bsum_ref[0]
    bhn = bhn_ref[0]
    f32 = jnp.float32
    bf = jnp.bfloat16

    a_bf = [a_ref[pl.ds(g * N, N), 0:N].astype(bf) for g in range(GB)]
    xf = x0_ref[...]
    for i in range(LAYERS):
        w = gw_ref[i]
        xb = xf.astype(bf)
        mf = jnp.dot(xb, w, preferred_element_type=f32)
        m3 = mf.astype(bf).reshape(GB, N, D)
        aggs = [jnp.dot(a_bf[g], m3[g], preferred_element_type=f32)
                for g in range(GB)]
        aggf = jnp.concatenate(aggs, axis=0)
        cat = jnp.concatenate([aggf, xf], axis=1).astype(bf)
        p = jnp.dot(cat, wcat, preferred_element_type=f32)
        hn = jnp.dot(xb, wnh, preferred_element_type=f32) + bhn
        r = _sig(p[:, 0:D] + bsum[0:D])
        z = _sig(p[:, 128:128 + D] + bsum[D:2 * D])
        n = jnp.tanh(p[:, 256:256 + D] + bsum[2 * D:] - (1.0 - r) * hn)
        xf = (1.0 - z) * n + z * xf

    x3 = xf.astype(bf).reshape(GB, N, D)
    for g in range(GB):
        ce = ce_ref[g]
        xg = x3[g]
        q = (ce[:, None] * cemb).astype(bf)
        sc = lax.dot_general(q, xg, (((1,), (1,)), ((), ())),
                             preferred_element_type=f32)
        ex = jnp.exp(sc)
        aw = (ex / jnp.sum(ex, axis=1, keepdims=True)).astype(bf)
        ao = jnp.dot(aw, xg, preferred_element_type=f32) * ce[:, None]
        nsum = jnp.sum(cid_ref[g])
        nsum = jnp.where(nsum == 0.0, 1.0, nsum)
        out_ref[g] = jnp.sum(ao, axis=0) / nsum


@jax.jit
def _tc_ggnn(x0, a3, ce, cid, cemb, gw, wcat, wnh, bsum, bhn):
    return pl.pallas_call(
        _ggnn_body,
        grid=(GRID1,),
        in_specs=[
            pl.BlockSpec((GB * N, D), lambda i: (i, 0)),
            pl.BlockSpec((GB * N, NP), lambda i: (i, 0)),
            pl.BlockSpec((GB, C1), lambda i: (i, 0)),
            pl.BlockSpec((GB, C1), lambda i: (i, 0)),
            pl.BlockSpec((C1, D), lambda i: (0, 0)),
            pl.BlockSpec((LAYERS, D, D), lambda i: (0, 0, 0)),
            pl.BlockSpec((2 * D, 384), lambda i: (0, 0)),
            pl.BlockSpec((D, D), lambda i: (0, 0)),
            pl.BlockSpec((1, 3 * D), lambda i: (0, 0)),
            pl.BlockSpec((1, D), lambda i: (0, 0)),
        ],
        out_specs=pl.BlockSpec((GB, D), lambda i: (i, 0)),
        out_shape=jax.ShapeDtypeStruct((B, D), jnp.float32),
        compiler_params=pltpu.CompilerParams(
            dimension_semantics=("arbitrary",)),
    )(x0, a3, ce, cid, cemb, gw, wcat, wnh, bsum, bhn)


def _lstm_body(ce_ref, at_ref, cr_ref, tc_ref, res_ref,
               wih_ref, whh_ref, bih_ref, bhh_ref, pw_ref, pb_ref,
               loss_ref, sig_ref, ft_ref, ihs_ref, hs_ref):
    f32 = jnp.float32
    ce3 = jnp.transpose(ce_ref[...], (1, 0, 2))
    at3 = jnp.transpose(at_ref[...].reshape(BS, SEQ, D), (1, 0, 2))
    cr3 = jnp.transpose(cr_ref[...], (1, 0, 2))
    xcat = jnp.concatenate([ce3, at3, cr3], axis=2)
    xflat = xcat.reshape(SEQ * BS, FEAT)
    ih = lax.dot_general(xflat, wih_ref[...], (((1,), (1,)), ((), ())),
                         preferred_element_type=f32) + bih_ref[0]
    ihs_ref[...] = ih.reshape(SEQ, BS, 4 * HID)
    whh = whh_ref[...]
    bhh = bhh_ref[0]

    def step(t, hc):
        h, c = hc
        g = ihs_ref[t] + lax.dot_general(
            h, whh, (((1,), (1,)), ((), ())), preferred_element_type=f32) + bhh
        i_g = g[:, 0:HID]
        f_g = g[:, HID:2 * HID]
        g_g = g[:, 2 * HID:3 * HID]
        o_g = g[:, 3 * HID:]
        c = _sig(f_g) * c + _sig(i_g) * jnp.tanh(g_g)
        h = _sig(o_g) * jnp.tanh(c)
        hs_ref[pl.ds(t, 1)] = h.reshape(1, BS, HID)
        return (h, c)

    lax.fori_loop(0, SEQ, step,
                  (jnp.zeros((BS, HID), f32), jnp.zeros((BS, HID), f32)))

    lo = hs_ref[...].reshape(SEQ * BS, HID)
    pred = lax.dot_general(lo, pw_ref[...], (((1,), (1,)), ((), ())),
                           preferred_element_type=f32) + pb_ref[0]
    tc = jnp.transpose(tc_ref[...], (1, 0, 2))
    p1 = jnp.sum(pred.reshape(SEQ, BS, C1) * tc, axis=2)
    numc = jnp.sum(tc, axis=2)
    mask = numc > 0.0
    safe = jnp.where(mask, numc, 1.0)
    fp = p1 / safe
    ft = res_ref[...]
    losses = (jnp.maximum(fp, 0.0) - fp * ft +
              jnp.log(1.0 + jnp.exp(-jnp.abs(fp))))
    cnt = jnp.sum(mask.astype(f32))
    loss = jnp.sum(jnp.where(mask, losses, 0.0)) / jnp.maximum(cnt, 1.0)
    loss_ref[...] = jnp.broadcast_to(loss, (1, 1))
    sig_ref[...] = _sig(jnp.where(mask, fp, 0.0))
    ft_ref[...] = jnp.where(mask, ft, 0.0)


@jax.jit
def _tc_lstm(ce_t, at_t, cr_t, tc_t, res_t, wih, whh, bih, bhh, pw, pb):
    return pl.pallas_call(
        _lstm_body,
        out_shape=[jax.ShapeDtypeStruct((1, 1), jnp.float32),
                   jax.ShapeDtypeStruct((SEQ, BS), jnp.float32),
                   jax.ShapeDtypeStruct((SEQ, BS), jnp.float32)],
        scratch_shapes=[pltpu.VMEM((SEQ, BS, 4 * HID), jnp.float32),
                        pltpu.VMEM((SEQ, BS, HID), jnp.float32)],
    )(ce_t, at_t, cr_t, tc_t, res_t, wih, whh, bih, bhh, pw, pb)


def kernel(p_id, c_id, node_id, edge, edge_type, target_c, result, c_embed,
           cur_result, node_embed_w, edge_embed_w, ggnn_w, gru_w_ih, gru_w_hh,
           gru_b_ih, gru_b_hh, lstm_w_ih, lstm_w_hh, lstm_b_ih, lstm_b_hh,
           pred_w, pred_b, concept_embedding):
    f32 = jnp.float32
    nid = node_id.reshape(ROWS).astype(jnp.int32)
    nid = jnp.concatenate(
        [nid, jnp.zeros((ROWS_PAD - ROWS,), jnp.int32)])
    src = edge[:, :, 0, :].reshape(B, E).astype(jnp.int32)
    dst = edge[:, :, 1, :].reshape(B, E).astype(jnp.int32)
    et = edge_type.reshape(B, E).astype(jnp.int32)

    x0_rows, a_mat = _sc_build(nid, node_embed_w.astype(f32), src, dst, et,
                               edge_embed_w.reshape(8 * D).astype(f32))

    bf = jnp.bfloat16
    wt = jnp.concatenate([gru_w_ih.T, gru_w_hh.T], axis=0)
    zpad = jnp.zeros((2 * D, D), f32)
    wcat = jnp.concatenate([wt[:, 0:D], zpad, wt[:, D:2 * D], zpad,
                            wt[:, 2 * D:], zpad], axis=1)
    wnh = gru_w_hh[2 * D:].T
    bsum = (gru_b_ih + gru_b_hh).reshape(1, 3 * D)
    bhn = gru_b_hh[2 * D:].reshape(1, D)
    attn = _tc_ggnn(x0_rows, a_mat, c_embed.reshape(B, C1).astype(f32),
                    c_id.reshape(B, C1).astype(f32),
                    concept_embedding.astype(bf), ggnn_w.astype(bf),
                    wcat.astype(bf), wnh.astype(bf),
                    bsum.astype(f32), bhn.astype(f32))

    res_t = result.reshape(BS, SEQ).T.astype(f32)

    loss2, sig2, ft2 = _tc_lstm(
        c_embed.astype(f32), attn, cur_result.astype(f32),
        target_c.astype(f32), res_t, lstm_w_ih.astype(f32),
        lstm_w_hh.astype(f32), lstm_b_ih.reshape(1, 4 * HID).astype(f32),
        lstm_b_hh.reshape(1, 4 * HID).astype(f32), pred_w.astype(f32),
        pred_b.reshape(1, C1).astype(f32))
    return (loss2.reshape(()), sig2.T.reshape(B), ft2.T.reshape(B))
